# baseline (speedup 1.0000x reference)
import functools

import jax
import jax.numpy as jnp
from jax import lax
from jax.experimental import pallas as pl
from jax.experimental.pallas import tpu as pltpu
from jax.experimental.pallas import tpu_sc as plsc

N = 50000
D = 1152
S = 512
BN = 1024
NW = 32
NP = 50176
CHUNK = NP // NW
G = NP // BN
SEGW = S * 8


def _head(xb, w1_ref, b1_ref, w2_ref, i):
    h = jnp.dot(xb, w1_ref[...], preferred_element_type=jnp.float32)
    h = jnp.tanh(h + b1_ref[...])
    return jnp.dot(h.astype(jnp.bfloat16), w2_ref[...],
                   preferred_element_type=jnp.float32)


def _mlp_body(x_ref, we_ref, wf_ref, ws_ref, be_ref, bf_ref, bs_ref,
              w2e_ref, w2f_ref, w2s_ref, b2_ref, osc_ref, f_ref):
    i = pl.program_id(0)
    xb = x_ref[...].astype(jnp.bfloat16)
    y = _head(xb, we_ref, be_ref, w2e_ref, i)
    y += _head(xb, wf_ref, bf_ref, w2f_ref, i)
    y += _head(xb, ws_ref, bs_ref, w2s_ref, i)
    y = y + b2_ref[...]
    rows = i * BN + lax.broadcasted_iota(jnp.int32, (BN, 16), 0)
    y = jnp.where(rows < N, y, 0.0)
    osc_ref[...] = y[:, 0:8]
    f_ref[...] = y[:, 8:11]


def _w1_spec():
    return pl.BlockSpec((D, D), lambda i: (0, 0))


def _b1_spec():
    return pl.BlockSpec((1, D), lambda i: (0, 0))


def _w2_spec():
    return pl.BlockSpec((D, 16), lambda i: (0, 0))


_mlp = pl.pallas_call(
    _mlp_body,
    grid=(G,),
    in_specs=[
        pl.BlockSpec((BN, D), lambda i: (i, 0)),
        _w1_spec(), _w1_spec(), _w1_spec(),
        _b1_spec(), _b1_spec(), _b1_spec(),
        _w2_spec(), _w2_spec(), _w2_spec(),
        pl.BlockSpec((1, 16), lambda i: (0, 0)),
    ],
    out_specs=[
        pl.BlockSpec((BN, 8), lambda i: (i, 0)),
        pl.BlockSpec((BN, 3), lambda i: (i, 0)),
    ],
    out_shape=[
        jax.ShapeDtypeStruct((NP, 8), jnp.float32),
        jax.ShapeDtypeStruct((N, 3), jnp.float32),
    ],
    compiler_params=pltpu.CompilerParams(dimension_semantics=("arbitrary",)),
)


def _seg_body(idx_hbm, vals_hbm, out_hbm, idx_v, vals_v, acc_v, part_v):
    wid = lax.axis_index("s") * 2 + lax.axis_index("c")
    pltpu.sync_copy(idx_hbm.at[pl.ds(wid * CHUNK, CHUNK)], idx_v)
    pltpu.sync_copy(vals_hbm.at[pl.ds(wid * CHUNK * 8, CHUNK * 8)], vals_v)

    zero16 = jnp.zeros((16,), jnp.float32)

    def zbody(i, c):
        acc_v[pl.ds(i * 16, 16)] = zero16
        return c

    lax.fori_loop(0, 16 * SEGW // 16, zbody, 0)

    iota = lax.iota(jnp.int32, 16)
    node_off = lax.shift_right_logical(iota, 3)
    col = jnp.bitwise_and(iota, 7)
    lane_base = iota * SEGW

    def body(g, c):
        nid = node_off + 2 * g
        seg = plsc.load_gather(idx_v, [nid])
        tgt = lane_base + seg * 8 + col
        val = vals_v[pl.ds(g * 16, 16)]
        plsc.addupdate_scatter(acc_v, [tgt], val)
        return c

    lax.fori_loop(0, CHUNK * 8 // 16, body, 0)

    def rbody(j, c):
        s = acc_v[pl.ds(j * 16, 16)]
        for l in range(1, 16):
            s = s + acc_v[pl.ds(l * SEGW + j * 16, 16)]
        part_v[pl.ds(j * 16, 16)] = s
        return c

    lax.fori_loop(0, SEGW // 16, rbody, 0)
    pltpu.sync_copy(part_v, out_hbm.at[wid])


@functools.lru_cache(maxsize=1)
def _get_seg_kernel():
    mesh = plsc.VectorSubcoreMesh(core_axis_name="c", subcore_axis_name="s")
    return functools.partial(
        pl.kernel,
        mesh=mesh,
        compiler_params=pltpu.CompilerParams(needs_layout_passes=False),
        out_type=jax.ShapeDtypeStruct((NW, SEGW), jnp.float32),
        scratch_types=[
            pltpu.VMEM((CHUNK,), jnp.int32),
            pltpu.VMEM((CHUNK * 8,), jnp.float32),
            pltpu.VMEM((16 * SEGW,), jnp.float32),
            pltpu.VMEM((SEGW,), jnp.float32),
        ],
    )(_seg_body)


def _red_body(p_ref, o_ref):
    o_ref[...] = jnp.sum(p_ref[...], axis=0, keepdims=True)


_reduce = pl.pallas_call(
    _red_body,
    out_shape=jax.ShapeDtypeStruct((1, SEGW), jnp.float32),
)


def _pad16(w, lo):
    out = jnp.zeros((D, 16), w.dtype)
    return lax.dynamic_update_slice(out, w, (0, lo)).astype(jnp.bfloat16)


def kernel(node_embedding, structure_index, We1, be1, We2, be2,
           Wf1, bf1, Wf2, bf2, Ws1, bs1, Ws2, bs2):
    w2e = _pad16(We2, 0)
    w2f = _pad16(Wf2, 8)
    w2s = _pad16(Ws2, 1)
    b2 = jnp.concatenate(
        [be2, bs2, jnp.zeros((1,), jnp.float32), bf2,
         jnp.zeros((5,), jnp.float32)]).reshape(1, 16)

    out_sc, forces = _mlp(
        node_embedding,
        We1.astype(jnp.bfloat16), Wf1.astype(jnp.bfloat16),
        Ws1.astype(jnp.bfloat16),
        be1.reshape(1, D), bf1.reshape(1, D), bs1.reshape(1, D),
        w2e, w2f, w2s, b2)

    idx_pad = jnp.pad(structure_index, (0, NP - N))
    parts = _get_seg_kernel()(idx_pad, out_sc.reshape(-1))
    red = _reduce(parts).reshape(S, 8)

    energy = red[:, 0]
    stress = red[:, 1:7]
    return (forces, energy, stress)


# speedup vs baseline: 1.0224x; 1.0224x over previous
import functools

import jax
import jax.numpy as jnp
from jax import lax
from jax.experimental import pallas as pl
from jax.experimental.pallas import tpu as pltpu
from jax.experimental.pallas import tpu_sc as plsc

N = 50000
D = 1152
S = 512
BN = 1024
NW = 32
NP = 50176
CHUNK = NP // NW
G = NP // BN
SEGW = S * 8


def _mlp_body(x_ref, w1_ref, b1_ref, w2_ref, b2_ref, osc_ref, f_ref):
    i = pl.program_id(0)
    xb = x_ref[...].astype(jnp.bfloat16)
    h = jnp.dot(xb, w1_ref[...], preferred_element_type=jnp.float32)
    h = jnp.tanh(h + b1_ref[...])
    y = jnp.dot(h.astype(jnp.bfloat16), w2_ref[...],
                preferred_element_type=jnp.float32)
    y = y + b2_ref[...]
    rows = i * BN + lax.broadcasted_iota(jnp.int32, (BN, 16), 0)
    y = jnp.where(rows < N, y, 0.0)
    osc_ref[...] = y[:, 0:8]
    f_ref[...] = y[:, 8:11]


_mlp = pl.pallas_call(
    _mlp_body,
    grid=(G,),
    in_specs=[
        pl.BlockSpec((BN, D), lambda i: (i, 0)),
        pl.BlockSpec((D, 3 * D), lambda i: (0, 0)),
        pl.BlockSpec((1, 3 * D), lambda i: (0, 0)),
        pl.BlockSpec((3 * D, 16), lambda i: (0, 0)),
        pl.BlockSpec((1, 16), lambda i: (0, 0)),
    ],
    out_specs=[
        pl.BlockSpec((BN, 8), lambda i: (i, 0)),
        pl.BlockSpec((BN, 3), lambda i: (i, 0)),
    ],
    out_shape=[
        jax.ShapeDtypeStruct((NP, 8), jnp.float32),
        jax.ShapeDtypeStruct((N, 3), jnp.float32),
    ],
    compiler_params=pltpu.CompilerParams(dimension_semantics=("arbitrary",)),
)


def _seg_body(idx_hbm, vals_hbm, out_hbm, idx_v, vals_v, acc_v, part_v):
    wid = lax.axis_index("s") * 2 + lax.axis_index("c")
    pltpu.sync_copy(idx_hbm.at[pl.ds(wid * CHUNK, CHUNK)], idx_v)
    pltpu.sync_copy(vals_hbm.at[pl.ds(wid * CHUNK * 8, CHUNK * 8)], vals_v)

    zero16 = jnp.zeros((16,), jnp.float32)

    def zbody(i, c):
        acc_v[pl.ds(i * 16, 16)] = zero16
        return c

    lax.fori_loop(0, 16 * SEGW // 16, zbody, 0)

    iota = lax.iota(jnp.int32, 16)
    node_off = lax.shift_right_logical(iota, 3)
    col = jnp.bitwise_and(iota, 7)
    lane_base = iota * SEGW

    def body(g, c):
        nid = node_off + 2 * g
        seg = plsc.load_gather(idx_v, [nid])
        tgt = lane_base + seg * 8 + col
        val = vals_v[pl.ds(g * 16, 16)]
        plsc.addupdate_scatter(acc_v, [tgt], val)
        return c

    lax.fori_loop(0, CHUNK * 8 // 16, body, 0)

    def rbody(j, c):
        s = acc_v[pl.ds(j * 16, 16)]
        for l in range(1, 16):
            s = s + acc_v[pl.ds(l * SEGW + j * 16, 16)]
        part_v[pl.ds(j * 16, 16)] = s
        return c

    lax.fori_loop(0, SEGW // 16, rbody, 0)
    pltpu.sync_copy(part_v, out_hbm.at[wid])


@functools.lru_cache(maxsize=1)
def _get_seg_kernel():
    mesh = plsc.VectorSubcoreMesh(core_axis_name="c", subcore_axis_name="s")
    return functools.partial(
        pl.kernel,
        mesh=mesh,
        compiler_params=pltpu.CompilerParams(needs_layout_passes=False),
        out_type=jax.ShapeDtypeStruct((NW, SEGW), jnp.float32),
        scratch_types=[
            pltpu.VMEM((CHUNK,), jnp.int32),
            pltpu.VMEM((CHUNK * 8,), jnp.float32),
            pltpu.VMEM((16 * SEGW,), jnp.float32),
            pltpu.VMEM((SEGW,), jnp.float32),
        ],
    )(_seg_body)


def _red_body(p_ref, o_ref):
    o_ref[...] = jnp.sum(p_ref[...], axis=0, keepdims=True)


_reduce = pl.pallas_call(
    _red_body,
    out_shape=jax.ShapeDtypeStruct((1, SEGW), jnp.float32),
)


def kernel(node_embedding, structure_index, We1, be1, We2, be2,
           Wf1, bf1, Wf2, bf2, Ws1, bs1, Ws2, bs2):
    w1 = jnp.concatenate([We1, Wf1, Ws1], axis=1).astype(jnp.bfloat16)
    b1 = jnp.concatenate([be1, bf1, bs1]).reshape(1, 3 * D)
    w2 = jnp.zeros((3 * D, 16), jnp.float32)
    w2 = w2.at[0:D, 0:1].set(We2)
    w2 = w2.at[D:2 * D, 8:11].set(Wf2)
    w2 = w2.at[2 * D:3 * D, 1:7].set(Ws2)
    w2 = w2.astype(jnp.bfloat16)
    b2 = jnp.concatenate(
        [be2, bs2, jnp.zeros((1,), jnp.float32), bf2,
         jnp.zeros((5,), jnp.float32)]).reshape(1, 16)

    out_sc, forces = _mlp(node_embedding, w1, b1, w2, b2)

    idx_pad = jnp.pad(structure_index, (0, NP - N))
    parts = _get_seg_kernel()(idx_pad, out_sc.reshape(-1))
    red = _reduce(parts).reshape(S, 8)

    energy = red[:, 0]
    stress = red[:, 1:7]
    return (forces, energy, stress)
